# baseline (speedup 1.0000x reference)
import jax
import jax.numpy as jnp
from jax.experimental import pallas as pl

_B, _N, _C, _D = 16, 1024, 384, 2048
_KA, _KB = 20, 4
_LAMBDA = 25.0
_MU = 25.0
_NU = 1.0
_ALPHA = 0.25
_EPS = 1e-4
_BIG = 3.0e38
_IDXMASK = 1023


def _pack(dist, iota_bits):
    di = jax.lax.bitcast_convert_type(dist, jnp.int32)
    return jax.lax.bitcast_convert_type((di & ~_IDXMASK) | iota_bits,
                                        jnp.float32)


def _unpack_idx(pmin):
    return jax.lax.bitcast_convert_type(pmin, jnp.int32) & _IDXMASK


def _phase1_kernel(za_ref, zb_ref, aga_ref, agbt_ref,
                   kc_ref, vc_ref, kr_ref, vr_ref):
    za = za_ref[0]
    zb = zb_ref[0]
    g = jax.lax.dot_general(za, zb, (((1,), (1,)), ((), ())),
                            preferred_element_type=jnp.float32,
                            precision=jax.lax.Precision.DEFAULT)
    x2 = jnp.sum(za * za, axis=1, keepdims=True)
    y2 = jnp.sum(zb * zb, axis=1, keepdims=True)
    y2r = jnp.reshape(y2, (1, _N))
    d2 = jnp.maximum(x2 + y2r - 2.0 * g, 0.0)
    rowmin_f = jnp.min(d2, axis=1, keepdims=True)
    colmin_f = jnp.min(d2, axis=0, keepdims=True)

    dg = jax.lax.dot_general(aga_ref[0], agbt_ref[0], (((1,), (0,)), ((), ())),
                             preferred_element_type=jnp.float32,
                             precision=jax.lax.Precision.DEFAULT)
    iota_j = jax.lax.broadcasted_iota(jnp.int32, (_N, _N), 1)
    iota_i = jax.lax.broadcasted_iota(jnp.int32, (_N, _N), 0)
    p_row = _pack(dg, iota_j)
    p_col = _pack(dg, iota_i)
    prow_min = jnp.min(p_row, axis=1, keepdims=True)
    pcol_min = jnp.min(p_col, axis=0, keepdims=True)
    row_arg = _unpack_idx(prow_min)
    col_arg = _unpack_idx(pcol_min)

    e_a = jnp.sum(jnp.where(iota_j == row_arg, d2, 0.0),
                  axis=1, keepdims=True)
    e_b = jnp.sum(jnp.where(iota_i == col_arg, d2, 0.0),
                  axis=0, keepdims=True)

    kc_ref[0, :, 0:1] = rowmin_f
    kc_ref[0, :, 1:2] = prow_min
    vc_ref[0, :, 0:1] = e_a
    kr_ref[0, 0:1, :] = colmin_f
    kr_ref[0, 1:2, :] = pcol_min
    vr_ref[0, 0:1, :] = e_b


def _phase2_kernel(kc_ref, vc_ref, kr_ref, vr_ref, zag_ref, zbg_ref, out_ref):
    kc = kc_ref[:, :, :]
    vc = vc_ref[:, :, :]

    def body_c(t, carry):
        ks, acc = carry
        m = jnp.min(ks, axis=1, keepdims=True)
        sel = ks == m
        ga = jnp.sum(jnp.where(sel[:, :, 1:2], vc, 0.0),
                     axis=1, keepdims=True)
        acc = acc + jnp.concatenate([m[:, :, 0:1], ga], axis=2)
        ks = jnp.where(sel, _BIG, ks)
        return ks, acc

    _, acc_c = jax.lax.fori_loop(
        0, _KA, body_c, (kc, jnp.zeros((_B, 1, 2), jnp.float32)))

    kr = kr_ref[:, :, :]
    vr = vr_ref[:, :, :]
    iota_sel = jax.lax.broadcasted_iota(jnp.int32, (1, 2, 1), 1)
    krow = jnp.where(iota_sel < 1, _KA, _KB)

    def body_r(t, carry):
        ks, acc = carry
        m = jnp.min(ks, axis=2, keepdims=True)
        sel = ks == m
        gb = jnp.sum(jnp.where(sel[:, 1:2, :], vr, 0.0),
                     axis=2, keepdims=True)
        contrib = jnp.concatenate([m[:, 0:1, :], gb], axis=1)
        w = (t < krow).astype(jnp.float32)
        acc = acc + contrib * w
        ks = jnp.where(sel, _BIG, ks)
        return ks, acc

    _, acc_r = jax.lax.fori_loop(
        0, _KA, body_r, (kr, jnp.zeros((_B, 2, 1), jnp.float32)))

    lcoef = _LAMBDA * (1.0 - _ALPHA) * 0.5
    c20 = lcoef / (_B * _KA * _C)
    c4 = lcoef / (_B * _KB * _C)
    local = (jnp.sum(acc_c) * c20 + jnp.sum(acc_r[:, 0:1, :]) * c20
             + jnp.sum(acc_r[:, 1:2, :]) * c4)

    zag = zag_ref[:, :]
    zbg = zbg_ref[:, :]
    inv_g = jnp.mean((zag - zbg) ** 2)

    def _var_cov(x):
        mu = jnp.mean(x, axis=0, keepdims=True)
        xc = x - mu
        var = jnp.sum(xc * xc, axis=0, keepdims=True) / (_B - 1)
        std = jnp.sqrt(var + _EPS)
        vloss = jnp.mean(jnp.maximum(1.0 - std, 0.0))
        a = jax.lax.dot_general(xc, xc, (((1,), (1,)), ((), ())),
                                preferred_element_type=jnp.float32,
                                precision=jax.lax.Precision.HIGHEST)
        frob = jnp.sum(a * a) / float((_B - 1) ** 2)
        closs = (frob - jnp.sum(var * var)) / _D
        return vloss, closs

    vl_a, cl_a = _var_cov(zag)
    vl_b, cl_b = _var_cov(zbg)
    global_loss = (_LAMBDA * inv_g + _MU * 0.5 * (vl_a + vl_b)
                   + _NU * (cl_a + cl_b))
    total = _ALPHA * global_loss + local
    out_ref[:, :] = total * jnp.ones((1, 1), jnp.float32)


def kernel(z_a, z_b, z_a_local_features, z_b_local_features, grid_a, grid_b):
    za_l = z_a_local_features.reshape(_B, _N, _C)
    zb_l = z_b_local_features.reshape(_B, _N, _C)
    ga = grid_a.reshape(_B, _N, 2)
    gb = grid_b.reshape(_B, _N, 2)

    ga_h = ga.astype(jnp.bfloat16).astype(jnp.float32)
    ga_l = ga - ga_h
    gb_h = gb.astype(jnp.bfloat16).astype(jnp.float32)
    gb_l = gb - gb_h
    ga2 = jnp.sum(ga * ga, axis=2, keepdims=True)
    gb2 = jnp.sum(gb * gb, axis=2, keepdims=True)
    ones = jnp.ones_like(ga2)
    aga = jnp.concatenate([ga_h, ga_h, ga_l, ones, ga2], axis=2)
    agb = jnp.concatenate([-2.0 * gb_h, -2.0 * gb_l, -2.0 * gb_h,
                           gb2, ones], axis=2)
    agbt = jnp.swapaxes(agb, 1, 2)

    kc, vc, kr, vr = pl.pallas_call(
        _phase1_kernel,
        grid=(_B,),
        in_specs=[
            pl.BlockSpec((1, _N, _C), lambda b: (b, 0, 0)),
            pl.BlockSpec((1, _N, _C), lambda b: (b, 0, 0)),
            pl.BlockSpec((1, _N, 8), lambda b: (b, 0, 0)),
            pl.BlockSpec((1, 8, _N), lambda b: (b, 0, 0)),
        ],
        out_specs=[
            pl.BlockSpec((1, _N, 2), lambda b: (b, 0, 0)),
            pl.BlockSpec((1, _N, 1), lambda b: (b, 0, 0)),
            pl.BlockSpec((1, 2, _N), lambda b: (b, 0, 0)),
            pl.BlockSpec((1, 1, _N), lambda b: (b, 0, 0)),
        ],
        out_shape=[
            jax.ShapeDtypeStruct((_B, _N, 2), jnp.float32),
            jax.ShapeDtypeStruct((_B, _N, 1), jnp.float32),
            jax.ShapeDtypeStruct((_B, 2, _N), jnp.float32),
            jax.ShapeDtypeStruct((_B, 1, _N), jnp.float32),
        ],
    )(za_l, zb_l, aga, agbt)

    out = pl.pallas_call(
        _phase2_kernel,
        out_shape=jax.ShapeDtypeStruct((1, 1), jnp.float32),
    )(kc, vc, kr, vr, z_a, z_b)
    return out.reshape(())
